# baseline (speedup 1.0000x reference)
import functools

import jax
import jax.numpy as jnp
from jax import lax
from jax.experimental import pallas as pl
from jax.experimental.pallas import tpu as pltpu

NEG_SLOPE = 0.01
BN_EPS = 1e-5


def _lrelu(v):
    return jnp.maximum(v, NEG_SLOPE * v)


def _tap_conv(a_ref, w_ref, b, *, taps, rows, row0=0):
    cout = w_ref.shape[2]

    acc = jnp.zeros((rows, cout), jnp.float32) + b
    for j in range(taps):
        sl = a_ref[pl.ds(row0 + j, rows), :]
        acc = acc + jnp.dot(sl, w_ref[j], preferred_element_type=jnp.float32)
    return acc


def _bn_lrelu(y, g, be, row_starts, nrows):
    c = y.shape[1]
    s1 = jnp.zeros((1, c), jnp.float32)
    s2 = jnp.zeros((1, c), jnp.float32)
    for r0 in row_starts:
        blk = y[r0:r0 + nrows]
        s1 = s1 + jnp.sum(blk, axis=0, keepdims=True)
        s2 = s2 + jnp.sum(blk * blk, axis=0, keepdims=True)
    inv_cnt = 1.0 / float(len(row_starts) * nrows)
    mean = s1 * inv_cnt
    var = jnp.maximum(s2 * inv_cnt - mean * mean, 0.0)
    scale = g * lax.rsqrt(var + BN_EPS)
    shift = be - mean * scale
    return _lrelu(y * scale + shift)


_B = 8
_RF = 372
_W2 = 353
_M2 = (_B - 1) * _RF + _W2
_W2q = 88
_R3F = 96
_S3 = 44
_M3 = (_B - 1) * _R3F + _S3
_W3q = 22
_R4F = 12
_W4 = 2
_M4 = (_B - 1) * _R4F + _W4
_YO = 768


def _conv_stack_kernel(p1_ref, w1_ref, b1_ref,
                       w2_ref, b2_ref, g2_ref, be2_ref,
                       w3e_ref, w3o_ref, b3_ref,
                       w4_ref, b4_ref, g4_ref, be4_ref,
                       o_ref, a1, a2, a3, ybuf, tmp):
    a2[...] = jnp.zeros_like(a2)
    a3[...] = jnp.zeros_like(a3)

    w1 = w1_ref[...]
    b1 = b1_ref[...]
    ya = jnp.dot(p1_ref[:, 0:200], w1, preferred_element_type=jnp.float32)
    yb = jnp.dot(p1_ref[:, 200:400], w1, preferred_element_type=jnp.float32)
    a1[:, 0:20] = _lrelu(ya + b1)
    a1[:, 20:40] = _lrelu(yb + b1)

    y = _tap_conv(a1, w2_ref, b2_ref[...], taps=20, rows=_M2)
    y = _bn_lrelu(y, g2_ref[...], be2_ref[...],
                  [b * _RF for b in range(_B)], _W2)
    ybuf[0:_M2, 0:40] = y
    for b in range(_B):
        r0 = b * _RF
        me = ybuf[pl.ds(r0 + 0, _S3, stride=8), 0:40]
        mo = ybuf[pl.ds(r0 + 4, _S3, stride=8), 0:40]
        for j in (1, 2, 3):
            me = jnp.maximum(me, ybuf[pl.ds(r0 + j, _S3, stride=8), 0:40])
            mo = jnp.maximum(mo, ybuf[pl.ds(r0 + 4 + j, _S3, stride=8), 0:40])
        a2[b * _R3F + 25:b * _R3F + 25 + _S3, 0:40] = me
        a2[b * _R3F + 25:b * _R3F + 25 + _S3, 40:80] = mo

    b3 = b3_ref[...]
    ye = _lrelu(_tap_conv(a2, w3e_ref, b3, taps=51, rows=_M3))
    yo = _lrelu(_tap_conv(a2, w3o_ref, b3, taps=50, rows=_M3, row0=1))
    ybuf[0:_M3, 0:80] = ye
    ybuf[_YO:_YO + _M3, 0:80] = yo
    for b in range(_B):
        r0 = b * _R3F
        m = jnp.maximum(ybuf[pl.ds(r0, 2 * _W3q), 0:80],
                        ybuf[pl.ds(_YO + r0, 2 * _W3q), 0:80])
        tmp[0:2 * _W3q, :] = m
        pe = jnp.maximum(tmp[pl.ds(0, 11, stride=4), :],
                         tmp[pl.ds(1, 11, stride=4), :])
        po = jnp.maximum(tmp[pl.ds(2, 11, stride=4), :],
                         tmp[pl.ds(3, 11, stride=4), :])
        a3[b * _R4F:b * _R4F + 11, 0:80] = pe
        a3[b * _R4F:b * _R4F + 11, 80:160] = po

    y = _tap_conv(a3, w4_ref, b4_ref[...], taps=10, rows=_M4)
    y = _bn_lrelu(y, g4_ref[...], be4_ref[...],
                  [b * _R4F for b in range(_B)], _W4)
    o_ref[...] = jnp.concatenate(
        [jnp.maximum(y[n * _R4F:n * _R4F + 1], y[n * _R4F + 1:n * _R4F + 2])
         for n in range(_B)], axis=0)


def _linear_kernel(f_ref, wl_ref, bl_ref, o_ref):
    o_ref[...] = jnp.dot(f_ref[...], wl_ref[...],
                         preferred_element_type=jnp.float32) + bl_ref[...]


def _logsoftmax_kernel(l_ref, o_ref):
    z = l_ref[...]
    z = z - jnp.max(z, axis=-1, keepdims=True)
    o_ref[...] = z - jnp.log(jnp.sum(jnp.exp(z), axis=-1, keepdims=True))


def _full_spec(shape):
    zeros = (0,) * len(shape)
    return pl.BlockSpec(tuple(shape), lambda *_, _z=zeros: _z)


def kernel(x, w1, b1, w2, b2, g2, be2, w3, b3, w4, b4, g4, be4, wl, bl):
    N, n_elec, records = x.shape
    ncls = wl.shape[1]

    xb = x.reshape(N * n_elec, 1, records, 1)
    p1 = lax.conv_general_dilated_patches(
        xb, filter_shape=(1, 200), window_strides=(1, 1),
        padding=((0, 0), (99, 100)),
        dimension_numbers=("NHWC", "HWIO", "NHWC"))
    p1f = p1.reshape(N * n_elec * records // 2, 400)

    w2f = w2.reshape(20, 40, 40)
    w3p = jnp.concatenate(
        [jnp.zeros((1, 40, 80), w3.dtype), w3, jnp.zeros((1, 40, 80), w3.dtype)])
    w3e = w3p.reshape(51, 80, 80)
    w3o = w3.reshape(50, 80, 80)
    w4f = w4.reshape(10, 160, 160)

    args = (p1f, w1, b1, w2f, b2, g2, be2, w3e, w3o, b3, w4f, b4, g4, be4)
    feats = pl.pallas_call(
        _conv_stack_kernel,
        out_shape=jax.ShapeDtypeStruct((_B, 160), jnp.float32),
        grid=(1,),
        in_specs=[_full_spec(a.shape) for a in args],
        out_specs=_full_spec((_B, 160)),
        scratch_shapes=[
            pltpu.VMEM((_B * _RF, 40), jnp.float32),
            pltpu.VMEM((_B * _R3F, 80), jnp.float32),
            pltpu.VMEM((_B * _R4F, 160), jnp.float32),
            pltpu.VMEM((2960, 80), jnp.float32),
            pltpu.VMEM((48, 80), jnp.float32),
        ],
        compiler_params=pltpu.CompilerParams(dimension_semantics=("arbitrary",)),
    )(*args)

    nb, cb = 6, ncls // 12
    logits = pl.pallas_call(
        _linear_kernel,
        out_shape=jax.ShapeDtypeStruct((N, ncls), jnp.float32),
        grid=(2, nb),
        in_specs=[
            pl.BlockSpec((_B, 160), lambda i, j: (0, 0)),
            pl.BlockSpec((160, cb), lambda i, j: (0, i * 6 + j)),
            pl.BlockSpec((1, cb), lambda i, j: (0, i * 6 + j)),
        ],
        out_specs=pl.BlockSpec((N, cb), lambda i, j: (0, i * 6 + j)),
        compiler_params=pltpu.CompilerParams(
            dimension_semantics=("parallel", "arbitrary")),
    )(feats, wl, bl)

    return pl.pallas_call(
        _logsoftmax_kernel,
        out_shape=jax.ShapeDtypeStruct((N, ncls), jnp.float32),
        grid=(1,),
        in_specs=[pl.BlockSpec((N, ncls), lambda i: (0, 0))],
        out_specs=pl.BlockSpec((N, ncls), lambda i: (0, 0)),
        compiler_params=pltpu.CompilerParams(
            dimension_semantics=("arbitrary",)),
    )(logits)
